# baseline (speedup 1.0000x reference)
import functools

import jax
import jax.numpy as jnp
from jax import lax
from jax.experimental import pallas as pl
from jax.experimental.pallas import tpu as pltpu
from jax.experimental.pallas import tpu_sc as plsc

N_NODES = 10000
N_EDGES = 320000
D = 128
DH = D // 2

NC = 2
NS = 16
NW = NC * NS

CH = 100
N_PAD = 10240
ROWS_PER_TILE = N_PAD // NS
ZROWS = 128

EDGE_NCHUNK = N_EDGES // NW // CH
FULL_NCHUNK = N_EDGES // NS // CH

_sc_mesh = plsc.VectorSubcoreMesh(core_axis_name="c", subcore_axis_name="s")
_sc_params = pltpu.CompilerParams(use_tc_tiling_on_sc=False)


@functools.partial(
    pl.kernel,
    out_type=jax.ShapeDtypeStruct((NC, N_PAD, 16), jnp.float32),
    mesh=_sc_mesh,
    scratch_types=[
        pltpu.VMEM((EDGE_NCHUNK, CH), jnp.int32),
        pltpu.VMEM((CH, 16), jnp.float32),
        pltpu.VMEM((ZROWS, 16), jnp.float32),
        pltpu.VMEM_SHARED((N_PAD, 16), jnp.float32),
    ],
    compiler_params=_sc_params,
)
def _sc_deg(dst_hbm, ones_hbm, zeros_hbm, out_hbm, dst_v, ones_v, zb_v, acc_sh):
    c = lax.axis_index("c")
    s = lax.axis_index("s")
    wid = s * NC + c

    pltpu.sync_copy(dst_hbm.at[wid], dst_v)
    pltpu.sync_copy(ones_hbm, ones_v)
    pltpu.sync_copy(zeros_hbm, zb_v)

    for k in range(ROWS_PER_TILE // ZROWS):
        pltpu.sync_copy(zb_v, acc_sh.at[pl.ds(s * ROWS_PER_TILE + k * ZROWS, ZROWS)])
    plsc.subcore_barrier()

    def body(j, carry):
        pltpu.sync_copy(ones_v, acc_sh.at[dst_v.at[j]], add=True)
        return carry

    lax.fori_loop(0, EDGE_NCHUNK, body, 0)
    plsc.subcore_barrier()

    r = s * ROWS_PER_TILE
    pltpu.sync_copy(acc_sh.at[pl.ds(r, ROWS_PER_TILE)],
                    out_hbm.at[c, pl.ds(r, ROWS_PER_TILE)])


@functools.partial(
    pl.kernel,
    out_type=jax.ShapeDtypeStruct((NC, N_PAD, DH), jnp.float32),
    mesh=_sc_mesh,
    scratch_types=[
        pltpu.VMEM((FULL_NCHUNK, CH), jnp.int32),
        pltpu.VMEM((FULL_NCHUNK, CH), jnp.int32),
        [pltpu.VMEM((CH, DH), jnp.float32)] * 6,
        pltpu.VMEM((ZROWS, DH), jnp.float32),
        pltpu.VMEM_SHARED((N_PAD, DH), jnp.float32),
        [pltpu.SemaphoreType.DMA] * 6,
        [pltpu.SemaphoreType.DMA] * 6,
    ],
    compiler_params=_sc_params,
)
def _sc_edge(src_hbm, dst_hbm, z_hbm, zeros_hbm, out_hbm,
             src_v, dst_v, bufs, zb_v, acc_sh, gsems, ssems):
    c = lax.axis_index("c")
    s = lax.axis_index("s")

    pltpu.sync_copy(src_hbm.at[s], src_v)
    pltpu.sync_copy(dst_hbm.at[s], dst_v)
    pltpu.sync_copy(zeros_hbm, zb_v)

    for k in range(ROWS_PER_TILE // ZROWS):
        pltpu.sync_copy(zb_v, acc_sh.at[pl.ds(s * ROWS_PER_TILE + k * ZROWS, ZROWS)])
    plsc.subcore_barrier()

    zc = z_hbm.at[c]

    def slot(j, b, b2):
        pltpu.make_async_copy(zc.at[src_v.at[j]], bufs[b], gsems[b]).wait()
        pltpu.async_copy(bufs[b], acc_sh.at[dst_v.at[j]], ssems[b], add=True)
        pltpu.make_async_copy(bufs[b2], acc_sh.at[dst_v.at[j - 2]],
                              ssems[b2]).wait()
        pltpu.async_copy(zc.at[src_v.at[j + 4]], bufs[b2], gsems[b2])

    for b in range(4):
        pltpu.async_copy(zc.at[src_v.at[b]], bufs[b], gsems[b])
    for j in range(2):
        pltpu.make_async_copy(zc.at[src_v.at[j]], bufs[j], gsems[j]).wait()
        pltpu.async_copy(bufs[j], acc_sh.at[dst_v.at[j]], ssems[j], add=True)
        pltpu.async_copy(zc.at[src_v.at[j + 4]], bufs[j + 4], gsems[j + 4])

    def body(g, carry):
        for bp in range(6):
            j = 6 * g + 2 + bp
            slot(j, (2 + bp) % 6, bp % 6)
        return carry

    _MAIN = (FULL_NCHUNK - 6) // 6
    lax.fori_loop(0, _MAIN, body, 0)
    for j in range(6 * _MAIN + 2, FULL_NCHUNK - 4):
        slot(j, j % 6, (j - 2) % 6)
    for j in range(FULL_NCHUNK - 4, FULL_NCHUNK):
        b = j % 6
        b2 = (j - 2) % 6
        pltpu.make_async_copy(zc.at[src_v.at[j]], bufs[b], gsems[b]).wait()
        pltpu.async_copy(bufs[b], acc_sh.at[dst_v.at[j]], ssems[b], add=True)
        pltpu.make_async_copy(bufs[b2], acc_sh.at[dst_v.at[j - 2]],
                              ssems[b2]).wait()
    for j in range(FULL_NCHUNK - 2, FULL_NCHUNK):
        b = j % 6
        pltpu.make_async_copy(bufs[b], acc_sh.at[dst_v.at[j]], ssems[b]).wait()

    plsc.subcore_barrier()

    r = s * ROWS_PER_TILE
    pltpu.sync_copy(acc_sh.at[pl.ds(r, ROWS_PER_TILE)],
                    out_hbm.at[c, pl.ds(r, ROWS_PER_TILE)])


_BLK = 1000
_GRID = N_NODES // _BLK


def _d_from_deg(deg_blk):
    deg = deg_blk[0, :, 0:1] + deg_blk[1, :, 0:1] + 1.0
    return lax.rsqrt(deg)


def _split_cols(arr_ref):
    return jnp.concatenate([arr_ref[0], arr_ref[1]], axis=1)


def _tc_mm_body(x_ref, w_ref, y_ref):
    y_ref[...] = jnp.dot(x_ref[...], w_ref[...],
                         preferred_element_type=jnp.float32)


def _tc_scale_body(y_ref, deg_ref, z_ref):
    d = _d_from_deg(deg_ref[...])
    z = d * y_ref[...]
    z_ref[0] = z[:, :DH]
    z_ref[1] = z[:, DH:]


def _tc_mid_body(z_ref, acc_ref, deg_ref, b_ref, w_ref, h_ref, zn_ref):
    d = _d_from_deg(deg_ref[...])
    tot = _split_cols(z_ref) + _split_cols(acc_ref)
    h = jnp.maximum(d * tot + b_ref[...], 0.0)
    h_ref[...] = h
    zn = d * jnp.dot(h, w_ref[...], preferred_element_type=jnp.float32)
    zn_ref[0] = zn[:, :DH]
    zn_ref[1] = zn[:, DH:]


_row_spec = pl.BlockSpec((_BLK, D), lambda i: (i, 0))
_half_spec = pl.BlockSpec((NC, _BLK, DH), lambda i: (0, i, 0))
_deg_spec = pl.BlockSpec((NC, _BLK, 16), lambda i: (0, i, 0))
_w_spec = pl.BlockSpec((D, D), lambda i: (0, 0))
_b_spec = pl.BlockSpec((1, D), lambda i: (0, 0))
_z_shape = jax.ShapeDtypeStruct((NC, N_NODES, DH), jnp.float32)
_h_shape = jax.ShapeDtypeStruct((N_NODES, D), jnp.float32)

_tc_mm = pl.pallas_call(
    _tc_mm_body,
    grid=(_GRID,),
    in_specs=[_row_spec, _w_spec],
    out_specs=_row_spec,
    out_shape=_h_shape,
)

_tc_scale = pl.pallas_call(
    _tc_scale_body,
    grid=(_GRID,),
    in_specs=[_row_spec, _deg_spec],
    out_specs=_half_spec,
    out_shape=_z_shape,
)

_tc_mid = pl.pallas_call(
    _tc_mid_body,
    grid=(_GRID,),
    in_specs=[_half_spec, _half_spec, _deg_spec, _b_spec, _w_spec],
    out_specs=(_row_spec, _half_spec),
    out_shape=(_h_shape, _z_shape),
)


def kernel(x, edge_index, W1, b1, W2, b2, W3, b3):
    ei = edge_index.astype(jnp.int32)
    src32 = ei[0].reshape(NW, EDGE_NCHUNK, CH)
    dst32 = ei[1].reshape(NW, EDGE_NCHUNK, CH)
    src16 = ei[0].reshape(NS, FULL_NCHUNK, CH)
    dst16 = ei[1].reshape(NS, FULL_NCHUNK, CH)

    ones16 = jnp.ones((CH, 16), jnp.float32)
    zeros16 = jnp.zeros((ZROWS, 16), jnp.float32)
    zerosH = jnp.zeros((ZROWS, DH), jnp.float32)

    degp = _sc_deg(dst32, ones16, zeros16)
    y1 = _tc_mm(x, W1)
    z1 = _tc_scale(y1, degp)

    bs = jnp.stack([b1, b2, b3]).reshape(3, 1, D)
    Ws = jnp.stack([W2, W3, W3])

    def layer(z, inputs):
        b, w = inputs
        acc = _sc_edge(src16, dst16, z, zerosH)
        h, zn = _tc_mid(z, acc, degp, b, w)
        return zn, h

    _, hs = lax.scan(layer, z1, (bs, Ws))
    return hs[2]


# speedup vs baseline: 1.0706x; 1.0706x over previous
import functools

import jax
import jax.numpy as jnp
from jax import lax
from jax.experimental import pallas as pl
from jax.experimental.pallas import tpu as pltpu
from jax.experimental.pallas import tpu_sc as plsc

N_NODES = 10000
N_EDGES = 320000
D = 128
DH = D // 2

NC = 2
NS = 16
NW = NC * NS

CH = 100
N_PAD = 10240
ROWS_PER_TILE = N_PAD // NS
ZROWS = 128

EDGE_NCHUNK = N_EDGES // NW // CH
FULL_NCHUNK = N_EDGES // NS // CH

_sc_mesh = plsc.VectorSubcoreMesh(core_axis_name="c", subcore_axis_name="s")
_sc_params = pltpu.CompilerParams(use_tc_tiling_on_sc=False)


@functools.partial(
    pl.kernel,
    out_type=jax.ShapeDtypeStruct((NC, N_PAD, 16), jnp.float32),
    mesh=_sc_mesh,
    scratch_types=[
        pltpu.VMEM((EDGE_NCHUNK, CH), jnp.int32),
        pltpu.VMEM((CH, 16), jnp.float32),
        pltpu.VMEM((ZROWS, 16), jnp.float32),
        pltpu.VMEM_SHARED((N_PAD, 16), jnp.float32),
    ],
    compiler_params=_sc_params,
)
def _sc_deg(dst_hbm, ones_hbm, zeros_hbm, out_hbm, dst_v, ones_v, zb_v, acc_sh):
    c = lax.axis_index("c")
    s = lax.axis_index("s")
    wid = s * NC + c

    pltpu.sync_copy(dst_hbm.at[wid], dst_v)
    pltpu.sync_copy(ones_hbm, ones_v)
    pltpu.sync_copy(zeros_hbm, zb_v)

    for k in range(ROWS_PER_TILE // ZROWS):
        pltpu.sync_copy(zb_v, acc_sh.at[pl.ds(s * ROWS_PER_TILE + k * ZROWS, ZROWS)])
    plsc.subcore_barrier()

    def body(j, carry):
        pltpu.sync_copy(ones_v, acc_sh.at[dst_v.at[j]], add=True)
        return carry

    lax.fori_loop(0, EDGE_NCHUNK, body, 0)
    plsc.subcore_barrier()

    r = s * ROWS_PER_TILE
    pltpu.sync_copy(acc_sh.at[pl.ds(r, ROWS_PER_TILE)],
                    out_hbm.at[c, pl.ds(r, ROWS_PER_TILE)])


@functools.partial(
    pl.kernel,
    out_type=jax.ShapeDtypeStruct((NC, N_PAD, DH), jnp.float32),
    mesh=_sc_mesh,
    scratch_types=[
        pltpu.VMEM((FULL_NCHUNK, CH), jnp.int32),
        pltpu.VMEM((FULL_NCHUNK, CH), jnp.int32),
        [pltpu.VMEM((CH, DH), jnp.float32)] * 6,
        pltpu.VMEM((ZROWS, DH), jnp.float32),
        pltpu.VMEM_SHARED((N_PAD, DH), jnp.float32),
        [pltpu.SemaphoreType.DMA] * 6,
        [pltpu.SemaphoreType.DMA] * 6,
    ],
    compiler_params=_sc_params,
)
def _sc_edge(src_hbm, dst_hbm, z_hbm, zeros_hbm, out_hbm,
             src_v, dst_v, bufs, zb_v, acc_sh, gsems, ssems):
    c = lax.axis_index("c")
    s = lax.axis_index("s")

    pltpu.sync_copy(src_hbm.at[s], src_v)
    pltpu.sync_copy(dst_hbm.at[s], dst_v)
    pltpu.sync_copy(zeros_hbm, zb_v)

    for k in range(ROWS_PER_TILE // ZROWS):
        pltpu.sync_copy(zb_v, acc_sh.at[pl.ds(s * ROWS_PER_TILE + k * ZROWS, ZROWS)])
    plsc.subcore_barrier()

    zc = z_hbm.at[c]

    def slot(j, b, b2):
        pltpu.make_async_copy(zc.at[src_v.at[j]], bufs[b], gsems[b]).wait()
        pltpu.async_copy(bufs[b], acc_sh.at[dst_v.at[j]], ssems[b], add=True)
        pltpu.make_async_copy(bufs[b2], acc_sh.at[dst_v.at[j - 2]],
                              ssems[b2]).wait()
        pltpu.async_copy(zc.at[src_v.at[j + 4]], bufs[b2], gsems[b2])

    for b in range(4):
        pltpu.async_copy(zc.at[src_v.at[b]], bufs[b], gsems[b])
    for j in range(2):
        pltpu.make_async_copy(zc.at[src_v.at[j]], bufs[j], gsems[j]).wait()
        pltpu.async_copy(bufs[j], acc_sh.at[dst_v.at[j]], ssems[j], add=True)
        pltpu.async_copy(zc.at[src_v.at[j + 4]], bufs[j + 4], gsems[j + 4])

    def body(g, carry):
        for bp in range(6):
            j = 6 * g + 2 + bp
            slot(j, (2 + bp) % 6, bp % 6)
        return carry

    _MAIN = (FULL_NCHUNK - 6) // 6
    lax.fori_loop(0, _MAIN, body, 0)
    for j in range(6 * _MAIN + 2, FULL_NCHUNK - 4):
        slot(j, j % 6, (j - 2) % 6)
    for j in range(FULL_NCHUNK - 4, FULL_NCHUNK):
        b = j % 6
        b2 = (j - 2) % 6
        pltpu.make_async_copy(zc.at[src_v.at[j]], bufs[b], gsems[b]).wait()
        pltpu.async_copy(bufs[b], acc_sh.at[dst_v.at[j]], ssems[b], add=True)
        pltpu.make_async_copy(bufs[b2], acc_sh.at[dst_v.at[j - 2]],
                              ssems[b2]).wait()
    for j in range(FULL_NCHUNK - 2, FULL_NCHUNK):
        b = j % 6
        pltpu.make_async_copy(bufs[b], acc_sh.at[dst_v.at[j]], ssems[b]).wait()

    plsc.subcore_barrier()

    r = s * ROWS_PER_TILE
    pltpu.sync_copy(acc_sh.at[pl.ds(r, ROWS_PER_TILE)],
                    out_hbm.at[c, pl.ds(r, ROWS_PER_TILE)])


_BLK = 2000
_GRID = N_NODES // _BLK


def _d_packed(deg_blk):
    cnt = deg_blk[0] + deg_blk[1] + 1.0
    d = lax.rsqrt(cnt)
    dp = d.reshape(_BLK // 2, 2, 16)
    de = dp[:, 0, :]
    do = dp[:, 1, :]
    return jnp.concatenate([de, de, de, de, do, do, do, do], axis=1)


def _tc_mm_body(x_ref, w_ref, y_ref):
    y_ref[...] = jnp.dot(x_ref[...], w_ref[...],
                         preferred_element_type=jnp.float32)


def _tc_scale_body(y_ref, deg_ref, z_ref):
    cnt = deg_ref[0][:, 0:1] + deg_ref[1][:, 0:1] + 1.0
    zy = lax.rsqrt(cnt) * y_ref[...]
    zy3 = zy.reshape(_BLK // 2, 2, D)
    z_ref[0] = jnp.concatenate([zy3[:, 0, :DH], zy3[:, 1, :DH]], axis=1)
    z_ref[1] = jnp.concatenate([zy3[:, 0, DH:], zy3[:, 1, DH:]], axis=1)


def _tc_mid_body(z_ref, acc_ref, deg_ref, b_ref, w_ref, h_ref, zn_ref):
    dpk = _d_packed(deg_ref[...])
    hlo = jnp.maximum(dpk * (z_ref[0] + acc_ref[0]) + b_ref[0], 0.0)
    hhi = jnp.maximum(dpk * (z_ref[1] + acc_ref[1]) + b_ref[1], 0.0)
    h_ref[0] = hlo
    h_ref[1] = hhi
    he = jnp.concatenate([hlo[:, :DH], hhi[:, :DH]], axis=1)
    ho = jnp.concatenate([hlo[:, DH:], hhi[:, DH:]], axis=1)
    ye = jnp.dot(he, w_ref[...], preferred_element_type=jnp.float32)
    yo = jnp.dot(ho, w_ref[...], preferred_element_type=jnp.float32)
    zn_ref[0] = dpk * jnp.concatenate([ye[:, :DH], yo[:, :DH]], axis=1)
    zn_ref[1] = dpk * jnp.concatenate([ye[:, DH:], yo[:, DH:]], axis=1)


def _tc_unpack_body(hp_ref, h_ref):
    he = jnp.concatenate([hp_ref[0][:, :DH], hp_ref[1][:, :DH]], axis=1)
    ho = jnp.concatenate([hp_ref[0][:, DH:], hp_ref[1][:, DH:]], axis=1)
    h_ref[...] = jnp.stack([he, ho], axis=1).reshape(_BLK, D)


_row_spec = pl.BlockSpec((_BLK, D), lambda i: (i, 0))
_pk_spec = pl.BlockSpec((NC, _BLK // 2, D), lambda i: (0, i, 0))
_deg_spec = pl.BlockSpec((NC, _BLK, 16), lambda i: (0, i, 0))
_w_spec = pl.BlockSpec((D, D), lambda i: (0, 0))
_bp_spec = pl.BlockSpec((NC, 1, D), lambda i: (0, 0, 0))
_zp_shape = jax.ShapeDtypeStruct((NC, N_NODES // 2, D), jnp.float32)
_h_shape = jax.ShapeDtypeStruct((N_NODES, D), jnp.float32)

_tc_mm = pl.pallas_call(
    _tc_mm_body,
    grid=(_GRID,),
    in_specs=[_row_spec, _w_spec],
    out_specs=_row_spec,
    out_shape=_h_shape,
)

_tc_scale = pl.pallas_call(
    _tc_scale_body,
    grid=(_GRID,),
    in_specs=[_row_spec, _deg_spec],
    out_specs=_pk_spec,
    out_shape=_zp_shape,
)

_tc_mid = pl.pallas_call(
    _tc_mid_body,
    grid=(_GRID,),
    in_specs=[_pk_spec, _pk_spec, _deg_spec, _bp_spec, _w_spec],
    out_specs=(_pk_spec, _pk_spec),
    out_shape=(_zp_shape, _zp_shape),
)

_tc_unpack = pl.pallas_call(
    _tc_unpack_body,
    grid=(_GRID,),
    in_specs=[_pk_spec],
    out_specs=_row_spec,
    out_shape=_h_shape,
)


def kernel(x, edge_index, W1, b1, W2, b2, W3, b3):
    ei = edge_index.astype(jnp.int32)
    src32 = ei[0].reshape(NW, EDGE_NCHUNK, CH)
    dst32 = ei[1].reshape(NW, EDGE_NCHUNK, CH)
    src16 = ei[0].reshape(NS, FULL_NCHUNK, CH)
    dst16 = ei[1].reshape(NS, FULL_NCHUNK, CH)

    ones16 = jnp.ones((CH, 16), jnp.float32)
    zeros16 = jnp.zeros((ZROWS, 16), jnp.float32)
    zerosH = jnp.zeros((ZROWS, DH), jnp.float32)

    degp = _sc_deg(dst32, ones16, zeros16)
    y1 = _tc_mm(x, W1)
    z1 = _tc_scale(y1, degp)

    def pack_b(b):
        return jnp.stack([jnp.concatenate([b[:DH], b[:DH]]),
                          jnp.concatenate([b[DH:], b[DH:]])]).reshape(NC, 1, D)

    bs = jnp.stack([pack_b(b1), pack_b(b2), pack_b(b3)])
    Ws = jnp.stack([W2, W3, W3])

    def layer(z, inputs):
        b, w = inputs
        acc = _sc_edge(src16, dst16, z.reshape(NC, N_NODES, DH), zerosH)
        h, zn = _tc_mid(z, acc.reshape(NC, N_PAD // 2, D), degp, b, w)
        return zn, h

    _, hs = lax.scan(layer, z1, (bs, Ws))
    return _tc_unpack(hs[2])


# speedup vs baseline: 1.0934x; 1.0213x over previous
import functools

import jax
import jax.numpy as jnp
from jax import lax
from jax.experimental import pallas as pl
from jax.experimental.pallas import tpu as pltpu
from jax.experimental.pallas import tpu_sc as plsc

N_NODES = 10000
N_EDGES = 320000
D = 128
DH = D // 2

NC = 2
NS = 16
NW = NC * NS

CH = 100
N_PAD = 10240
ROWS_PER_TILE = N_PAD // NS
ZROWS = 128

EDGE_NCHUNK = N_EDGES // NW // CH
FULL_NCHUNK = N_EDGES // NS // CH

_sc_mesh = plsc.VectorSubcoreMesh(core_axis_name="c", subcore_axis_name="s")
_sc_params = pltpu.CompilerParams(use_tc_tiling_on_sc=False)


@functools.partial(
    pl.kernel,
    out_type=jax.ShapeDtypeStruct((NC, N_PAD, 16), jnp.float32),
    mesh=_sc_mesh,
    scratch_types=[
        pltpu.VMEM((EDGE_NCHUNK, CH), jnp.int32),
        pltpu.VMEM((CH, 16), jnp.float32),
        pltpu.VMEM((ZROWS, 16), jnp.float32),
        pltpu.VMEM_SHARED((N_PAD, 16), jnp.float32),
    ],
    compiler_params=_sc_params,
)
def _sc_deg(dst_hbm, ones_hbm, zeros_hbm, out_hbm, dst_v, ones_v, zb_v, acc_sh):
    c = lax.axis_index("c")
    s = lax.axis_index("s")
    wid = s * NC + c

    pltpu.sync_copy(dst_hbm.at[wid], dst_v)
    pltpu.sync_copy(ones_hbm, ones_v)
    pltpu.sync_copy(zeros_hbm, zb_v)

    for k in range(ROWS_PER_TILE // ZROWS):
        pltpu.sync_copy(zb_v, acc_sh.at[pl.ds(s * ROWS_PER_TILE + k * ZROWS, ZROWS)])
    plsc.subcore_barrier()

    def body(j, carry):
        pltpu.sync_copy(ones_v, acc_sh.at[dst_v.at[j]], add=True)
        return carry

    lax.fori_loop(0, EDGE_NCHUNK, body, 0)
    plsc.subcore_barrier()

    r = s * ROWS_PER_TILE
    pltpu.sync_copy(acc_sh.at[pl.ds(r, ROWS_PER_TILE)],
                    out_hbm.at[c, pl.ds(r, ROWS_PER_TILE)])


@functools.partial(
    pl.kernel,
    out_type=jax.ShapeDtypeStruct((NC, N_PAD, DH), jnp.float32),
    mesh=_sc_mesh,
    scratch_types=[
        pltpu.VMEM((FULL_NCHUNK, CH), jnp.int32),
        pltpu.VMEM((FULL_NCHUNK, CH), jnp.int32),
        [pltpu.VMEM((CH, DH), jnp.float32)] * 6,
        pltpu.VMEM((ZROWS, DH), jnp.float32),
        pltpu.VMEM_SHARED((N_PAD, DH), jnp.float32),
        [pltpu.SemaphoreType.DMA] * 6,
        [pltpu.SemaphoreType.DMA] * 6,
    ],
    compiler_params=_sc_params,
)
def _sc_edge(src_hbm, dst_hbm, z_hbm, zeros_hbm, out_hbm,
             src_v, dst_v, bufs, zb_v, acc_sh, gsems, ssems):
    c = lax.axis_index("c")
    s = lax.axis_index("s")

    pltpu.sync_copy(src_hbm.at[s], src_v)
    pltpu.sync_copy(dst_hbm.at[s], dst_v)
    pltpu.sync_copy(zeros_hbm, zb_v)

    for k in range(ROWS_PER_TILE // ZROWS):
        pltpu.sync_copy(zb_v, acc_sh.at[pl.ds(s * ROWS_PER_TILE + k * ZROWS, ZROWS)])
    plsc.subcore_barrier()

    zc = z_hbm.at[c]

    def slot(j, b, b2):
        pltpu.make_async_copy(zc.at[src_v.at[j]], bufs[b], gsems[b]).wait()
        pltpu.async_copy(bufs[b], acc_sh.at[dst_v.at[j]], ssems[b], add=True)
        pltpu.make_async_copy(bufs[b2], acc_sh.at[dst_v.at[j - 2]],
                              ssems[b2]).wait()
        pltpu.async_copy(zc.at[src_v.at[j + 4]], bufs[b2], gsems[b2])

    for b in range(4):
        pltpu.async_copy(zc.at[src_v.at[b]], bufs[b], gsems[b])
    for j in range(2):
        pltpu.make_async_copy(zc.at[src_v.at[j]], bufs[j], gsems[j]).wait()
        pltpu.async_copy(bufs[j], acc_sh.at[dst_v.at[j]], ssems[j], add=True)
        pltpu.async_copy(zc.at[src_v.at[j + 4]], bufs[j + 4], gsems[j + 4])

    def body(g, carry):
        for bp in range(6):
            j = 6 * g + 2 + bp
            slot(j, (2 + bp) % 6, bp % 6)
        return carry

    _MAIN = (FULL_NCHUNK - 6) // 6
    lax.fori_loop(0, _MAIN, body, 0)
    for j in range(6 * _MAIN + 2, FULL_NCHUNK - 4):
        slot(j, j % 6, (j - 2) % 6)
    for j in range(FULL_NCHUNK - 4, FULL_NCHUNK):
        b = j % 6
        b2 = (j - 2) % 6
        pltpu.make_async_copy(zc.at[src_v.at[j]], bufs[b], gsems[b]).wait()
        pltpu.async_copy(bufs[b], acc_sh.at[dst_v.at[j]], ssems[b], add=True)
        pltpu.make_async_copy(bufs[b2], acc_sh.at[dst_v.at[j - 2]],
                              ssems[b2]).wait()
    for j in range(FULL_NCHUNK - 2, FULL_NCHUNK):
        b = j % 6
        pltpu.make_async_copy(bufs[b], acc_sh.at[dst_v.at[j]], ssems[b]).wait()

    plsc.subcore_barrier()

    r = s * ROWS_PER_TILE
    pltpu.sync_copy(acc_sh.at[pl.ds(r, ROWS_PER_TILE)],
                    out_hbm.at[c, pl.ds(r, ROWS_PER_TILE)])


_BLK = 2000
_GRID = N_NODES // _BLK


def _tc_dpk_body(deg_ref, dpk_ref):
    cnt = deg_ref[0] + deg_ref[1] + 1.0
    d = lax.rsqrt(cnt)
    dp = d.reshape(_BLK // 2, 2, 16)
    de = dp[:, 0, :]
    do = dp[:, 1, :]
    dpk_ref[...] = jnp.concatenate([de, de, de, de, do, do, do, do], axis=1)


def _tc_mm_body(x_ref, w_ref, y_ref):
    y_ref[...] = jnp.dot(x_ref[...], w_ref[...],
                         preferred_element_type=jnp.float32)


def _tc_scale_body(y_ref, dpk_ref, z_ref):
    dpk = dpk_ref[...]
    de = dpk[:, 0:1]
    do = dpk[:, DH:DH + 1]
    y3 = y_ref[...].reshape(_BLK // 2, 2, D)
    ye = de * y3[:, 0, :]
    yo = do * y3[:, 1, :]
    z_ref[0] = jnp.concatenate([ye[:, :DH], yo[:, :DH]], axis=1)
    z_ref[1] = jnp.concatenate([ye[:, DH:], yo[:, DH:]], axis=1)


def _tc_mid_body(z_ref, acc_ref, dpk_ref, b_ref, w_ref, h_ref, zn_ref):
    dpk = dpk_ref[...]
    hlo = jnp.maximum(dpk * (z_ref[0] + acc_ref[0]) + b_ref[0], 0.0)
    hhi = jnp.maximum(dpk * (z_ref[1] + acc_ref[1]) + b_ref[1], 0.0)
    h_ref[0] = hlo
    h_ref[1] = hhi
    he = jnp.concatenate([hlo[:, :DH], hhi[:, :DH]], axis=1)
    ho = jnp.concatenate([hlo[:, DH:], hhi[:, DH:]], axis=1)
    ye = jnp.dot(he, w_ref[...], preferred_element_type=jnp.float32)
    yo = jnp.dot(ho, w_ref[...], preferred_element_type=jnp.float32)
    zn_ref[0] = dpk * jnp.concatenate([ye[:, :DH], yo[:, :DH]], axis=1)
    zn_ref[1] = dpk * jnp.concatenate([ye[:, DH:], yo[:, DH:]], axis=1)


def _tc_unpack_body(hp_ref, h_ref):
    he = jnp.concatenate([hp_ref[0][:, :DH], hp_ref[1][:, :DH]], axis=1)
    ho = jnp.concatenate([hp_ref[0][:, DH:], hp_ref[1][:, DH:]], axis=1)
    h_ref[...] = jnp.stack([he, ho], axis=1).reshape(_BLK, D)


_row_spec = pl.BlockSpec((_BLK, D), lambda i: (i, 0))
_pk_spec = pl.BlockSpec((NC, _BLK // 2, D), lambda i: (0, i, 0))
_dpk_spec = pl.BlockSpec((_BLK // 2, D), lambda i: (i, 0))
_deg_spec = pl.BlockSpec((NC, _BLK, 16), lambda i: (0, i, 0))
_w_spec = pl.BlockSpec((D, D), lambda i: (0, 0))
_bp_spec = pl.BlockSpec((NC, 1, D), lambda i: (0, 0, 0))
_zp_shape = jax.ShapeDtypeStruct((NC, N_NODES // 2, D), jnp.float32)
_dpk_shape = jax.ShapeDtypeStruct((N_NODES // 2, D), jnp.float32)
_h_shape = jax.ShapeDtypeStruct((N_NODES, D), jnp.float32)

_tc_dpk = pl.pallas_call(
    _tc_dpk_body,
    grid=(_GRID,),
    in_specs=[_deg_spec],
    out_specs=_dpk_spec,
    out_shape=_dpk_shape,
)

_tc_mm = pl.pallas_call(
    _tc_mm_body,
    grid=(_GRID,),
    in_specs=[_row_spec, _w_spec],
    out_specs=_row_spec,
    out_shape=_h_shape,
)

_tc_scale = pl.pallas_call(
    _tc_scale_body,
    grid=(_GRID,),
    in_specs=[_row_spec, _dpk_spec],
    out_specs=_pk_spec,
    out_shape=_zp_shape,
)

_tc_mid = pl.pallas_call(
    _tc_mid_body,
    grid=(_GRID,),
    in_specs=[_pk_spec, _pk_spec, _dpk_spec, _bp_spec, _w_spec],
    out_specs=(_pk_spec, _pk_spec),
    out_shape=(_zp_shape, _zp_shape),
)

_tc_unpack = pl.pallas_call(
    _tc_unpack_body,
    grid=(_GRID,),
    in_specs=[_pk_spec],
    out_specs=_row_spec,
    out_shape=_h_shape,
)


def kernel(x, edge_index, W1, b1, W2, b2, W3, b3):
    ei = edge_index.astype(jnp.int32)
    src32 = ei[0].reshape(NW, EDGE_NCHUNK, CH)
    dst32 = ei[1].reshape(NW, EDGE_NCHUNK, CH)
    src16 = ei[0].reshape(NS, FULL_NCHUNK, CH)
    dst16 = ei[1].reshape(NS, FULL_NCHUNK, CH)

    ones16 = jnp.ones((CH, 16), jnp.float32)
    zeros16 = jnp.zeros((ZROWS, 16), jnp.float32)
    zerosH = jnp.zeros((ZROWS, DH), jnp.float32)

    degp = _sc_deg(dst32, ones16, zeros16)
    y1 = _tc_mm(x, W1)
    dpk = _tc_dpk(degp)
    z1 = _tc_scale(y1, dpk)

    def pack_b(b):
        return jnp.stack([jnp.concatenate([b[:DH], b[:DH]]),
                          jnp.concatenate([b[DH:], b[DH:]])]).reshape(NC, 1, D)

    bs = jnp.stack([pack_b(b1), pack_b(b2), pack_b(b3)])
    Ws = jnp.stack([W2, W3, W3])

    def layer(z, inputs):
        b, w = inputs
        acc = _sc_edge(src16, dst16, z.reshape(NC, N_NODES, DH), zerosH)
        h, zn = _tc_mid(z, acc.reshape(NC, N_PAD // 2, D), dpk, b, w)
        return zn, h

    _, hs = lax.scan(layer, z1, (bs, Ws))
    return _tc_unpack(hs[2])
